# baseline (speedup 1.0000x reference)
import functools

import jax
import jax.numpy as jnp
from jax import lax
from jax.experimental import pallas as pl
from jax.experimental.pallas import tpu as pltpu
from jax.experimental.pallas import tpu_sc as plsc

_VOCAB = 100000
_D = 16
_B = 16384
_NC = 2
_NS = 16
_NW = _NC * _NS
_BPW = _B // _NW
_REC = 1 + _D


@functools.partial(
    pl.kernel,
    mesh=plsc.VectorSubcoreMesh(core_axis_name="c", subcore_axis_name="s"),
    out_type=jax.ShapeDtypeStruct((_B * _REC,), jnp.float32),
    compiler_params=pltpu.CompilerParams(use_tc_tiling_on_sc=False),
    scratch_types=[
        pltpu.VMEM((_BPW,), jnp.int32),
        pltpu.VMEM((_BPW, _D), jnp.float32),
        pltpu.VMEM((_BPW,), jnp.float32),
        pltpu.VMEM((_BPW * _REC,), jnp.float32),
        pltpu.SemaphoreType.DMA,
    ],
)
def _emb_concat(pitch_hbm, idx_hbm, table_hbm, out_hbm, idx_v, rows_v,
                pitch_v, out_v, sem):
    wid = lax.axis_index("s") * _NC + lax.axis_index("c")
    base = wid * _BPW
    pltpu.sync_copy(idx_hbm.at[pl.ds(base, _BPW)], idx_v)
    pltpu.sync_copy(pitch_hbm.at[pl.ds(base, _BPW)], pitch_v)
    pltpu.async_copy(table_hbm.at[idx_v], rows_v, sem).wait()

    def body(g, _):
        row0 = g * 16
        p = pitch_v[pl.ds(row0, 16)]
        for r in range(16):
            i = row0 + r
            out_v[pl.ds(i * _REC, 16)] = jnp.full((16,), p[r],
                                                  dtype=jnp.float32)
            out_v[pl.ds(i * _REC + 1, _D)] = rows_v[i, :]
        return 0

    lax.fori_loop(0, _BPW // 16, body, 0)
    pltpu.sync_copy(out_v, out_hbm.at[pl.ds(base * _REC, _BPW * _REC)])


def kernel(pitch, timbre_id, table):
    return _emb_concat(pitch, timbre_id, table).reshape(_B, _REC)


# speedup vs baseline: 2.1238x; 2.1238x over previous
import functools

import jax
import jax.numpy as jnp
from jax import lax
from jax.experimental import pallas as pl
from jax.experimental.pallas import tpu as pltpu
from jax.experimental.pallas import tpu_sc as plsc

_VOCAB = 100000
_D = 16
_B = 16384
_NC = 2
_NS = 16
_NW = _NC * _NS
_BPW = _B // _NW
_REC = 1 + _D


@functools.partial(
    pl.kernel,
    mesh=plsc.VectorSubcoreMesh(core_axis_name="c", subcore_axis_name="s"),
    out_type=jax.ShapeDtypeStruct((_REC, _B), jnp.float32),
    compiler_params=pltpu.CompilerParams(use_tc_tiling_on_sc=False),
    scratch_types=[
        pltpu.VMEM((_BPW,), jnp.int32),
        pltpu.VMEM((_REC, _BPW), jnp.float32),
        pltpu.SemaphoreType.DMA,
    ],
)
def _emb_concat(pitch_hbm, idx_hbm, tab_t_hbm, out_hbm, idx_v, out_v, sem):
    wid = lax.axis_index("s") * _NC + lax.axis_index("c")
    base = wid * _BPW
    pltpu.sync_copy(idx_hbm.at[pl.ds(base, _BPW)], idx_v)
    pltpu.sync_copy(pitch_hbm.at[pl.ds(base, _BPW)], out_v.at[0])
    copies = [
        pltpu.async_copy(tab_t_hbm.at[c].at[idx_v], out_v.at[1 + c], sem)
        for c in range(_D)
    ]
    for cp in copies:
        cp.wait()
    pltpu.sync_copy(out_v, out_hbm.at[:, pl.ds(base, _BPW)])


def kernel(pitch, timbre_id, table):
    return _emb_concat(pitch, timbre_id, table.T).T


# speedup vs baseline: 2.1560x; 1.0151x over previous
import functools

import jax
import jax.numpy as jnp
from jax import lax
from jax.experimental import pallas as pl
from jax.experimental.pallas import tpu as pltpu
from jax.experimental.pallas import tpu_sc as plsc

_VOCAB = 100000
_D = 16
_B = 16384
_NC = 2
_NS = 16
_NW = _NC * _NS
_BPW = _B // _NW
_REC = 1 + _D


@functools.partial(
    pl.kernel,
    mesh=plsc.VectorSubcoreMesh(core_axis_name="c", subcore_axis_name="s"),
    out_type=jax.ShapeDtypeStruct((_REC, _B), jnp.float32),
    compiler_params=pltpu.CompilerParams(use_tc_tiling_on_sc=False),
    scratch_types=[
        pltpu.VMEM((_BPW,), jnp.int32),
        pltpu.VMEM((_REC, _BPW), jnp.float32),
        pltpu.SemaphoreType.DMA,
        pltpu.SemaphoreType.DMA,
    ],
)
def _emb_concat(pitch_hbm, idx_hbm, tab_t_hbm, out_hbm, idx_v, out_v, sem,
                osem):
    wid = lax.axis_index("s") * _NC + lax.axis_index("c")
    base = wid * _BPW
    pltpu.sync_copy(idx_hbm.at[pl.ds(base, _BPW)], idx_v)
    gathers = [
        pltpu.async_copy(tab_t_hbm.at[c].at[idx_v], out_v.at[1 + c], sem)
        for c in range(_D)
    ]
    pltpu.sync_copy(pitch_hbm.at[pl.ds(base, _BPW)], out_v.at[0])
    writes = [
        pltpu.async_copy(out_v.at[0], out_hbm.at[0, pl.ds(base, _BPW)], osem)
    ]
    for c in range(_D):
        gathers[c].wait()
        writes.append(
            pltpu.async_copy(out_v.at[1 + c],
                             out_hbm.at[1 + c, pl.ds(base, _BPW)], osem))
    for w in writes:
        w.wait()


def kernel(pitch, timbre_id, table):
    return _emb_concat(pitch, timbre_id, table.T).T
